# Initial kernel scaffold; baseline (speedup 1.0000x reference)
import jax
import jax.numpy as jnp
from jax.experimental import pallas as pl


def kernel(pitch_type, table):
    raise NotImplementedError("write your pallas kernel here")


# speedup vs baseline: 2.3604x; 2.3604x over previous
import jax
import jax.numpy as jnp
from jax.experimental import pallas as pl
from jax.experimental.pallas import tpu as pltpu
from jax.experimental.pallas import tpu_sc as plsc

_BATCH = 16384
_HIST = 200
_EMBED = 16
_N = _BATCH * _HIST

_WINDOW = 2048


def kernel(pitch_type, table):
    idx = pitch_type.reshape(1, _N).astype(jnp.int32)

    vector_mesh = plsc.VectorSubcoreMesh(
        core_axis_name="core", subcore_axis_name="subcore"
    )

    @pl.kernel(
        out_type=jax.ShapeDtypeStruct((_N, _EMBED), jnp.float32),
        mesh=vector_mesh,
        compiler_params=pltpu.CompilerParams(use_tc_tiling_on_sc=False),
    )
    def gather_kernel(x_hbm, i_hbm, o_hbm):
        def body(i_vmem, o_vmem):
            pltpu.sync_copy(x_hbm.at[i_vmem.at[0]], o_vmem)

        pltpu.emit_pipeline(
            body,
            grid=(_N // _WINDOW,),
            in_specs=[
                pl.BlockSpec((1, _WINDOW), index_map=lambda i: (0, i))
            ],
            out_specs=[
                pl.BlockSpec((_WINDOW, _EMBED), index_map=lambda i: (i, 0))
            ],
            core_axis_name="subcore",
            dimension_semantics=(pltpu.PARALLEL,),
        )(i_hbm, o_hbm)

    out = gather_kernel(table, idx)
    return out.reshape(_BATCH, _HIST, _EMBED)


# speedup vs baseline: 2.5321x; 1.0728x over previous
import jax
import jax.numpy as jnp
from jax.experimental import pallas as pl
from jax.experimental.pallas import tpu as pltpu
from jax.experimental.pallas import tpu_sc as plsc

_BATCH = 16384
_HIST = 200
_EMBED = 16
_N = _BATCH * _HIST

_WINDOW = 2048


def kernel(pitch_type, table):
    idx = pitch_type.reshape(1, _N).astype(jnp.int32)

    vector_mesh = plsc.VectorSubcoreMesh(
        core_axis_name="core", subcore_axis_name="subcore"
    )

    @pl.kernel(
        out_type=jax.ShapeDtypeStruct((_N, _EMBED), jnp.float32),
        mesh=vector_mesh,
        compiler_params=pltpu.CompilerParams(use_tc_tiling_on_sc=False),
    )
    def gather_kernel(x_hbm, i_hbm, o_hbm):
        core_id = jax.lax.axis_index("core")
        steps_per_core = _N // _WINDOW // 2
        step0 = core_id * steps_per_core

        def body(i_vmem, o_vmem):
            pltpu.sync_copy(x_hbm.at[i_vmem.at[0]], o_vmem)

        pltpu.emit_pipeline(
            body,
            grid=(steps_per_core,),
            in_specs=[
                pl.BlockSpec((1, _WINDOW), index_map=lambda i: (0, step0 + i))
            ],
            out_specs=[
                pl.BlockSpec((_WINDOW, _EMBED), index_map=lambda i: (step0 + i, 0))
            ],
            core_axis_name="subcore",
            dimension_semantics=(pltpu.PARALLEL,),
        )(i_hbm, o_hbm)

    out = gather_kernel(table, idx)
    return out.reshape(_BATCH, _HIST, _EMBED)


# speedup vs baseline: 4.2256x; 1.6688x over previous
import jax
import jax.numpy as jnp
from jax import lax
from jax.experimental import pallas as pl
from jax.experimental.pallas import tpu as pltpu
from jax.experimental.pallas import tpu_sc as plsc

_BATCH = 16384
_HIST = 200
_EMBED = 16
_N = _BATCH * _HIST

_HT = _HIST // 8
_BT = _BATCH // 128
_TILES = _HT * _BT


def kernel(pitch_type, table):
    idx_phys = (
        pitch_type.T.reshape(_HT, 8, _BT, 128)
        .transpose(0, 2, 1, 3)
        .reshape(_TILES, 1024)
        .astype(jnp.int32)
    )

    vector_mesh = plsc.VectorSubcoreMesh(
        core_axis_name="core", subcore_axis_name="subcore"
    )

    @pl.kernel(
        out_type=jax.ShapeDtypeStruct((_HIST, 2, _BT, 8, 128), jnp.float32),
        mesh=vector_mesh,
        scratch_types=[pltpu.VMEM((1024, _EMBED), jnp.float32)],
        compiler_params=pltpu.CompilerParams(
            use_tc_tiling_on_sc=False, needs_layout_passes=False
        ),
    )
    def gather_kernel(x_hbm, i_hbm, o_hbm, rows):
        core_id = lax.axis_index("core")
        steps_per_core = _TILES // 2
        step0 = core_id * steps_per_core
        iota16 = lax.iota(jnp.int32, 16)

        def body(i_vmem, o_vmem):
            pltpu.sync_copy(x_hbm.at[i_vmem.at[0]], rows)

            def g_body(g, carry):
                h = g // 8
                b0 = (g % 8) * 16
                row_ids = g * 16 + iota16
                for c in range(_EMBED):
                    col_ids = jnp.full((16,), c, jnp.int32)
                    vec = plsc.load_gather(rows, [row_ids, col_ids])
                    o_vmem[h, c // 8, 0, c % 8, pl.ds(b0, 16)] = vec
                return carry

            lax.fori_loop(0, 64, g_body, 0)

        pltpu.emit_pipeline(
            body,
            grid=(steps_per_core,),
            in_specs=[
                pl.BlockSpec((1, 1024), index_map=lambda i: (step0 + i, 0))
            ],
            out_specs=[
                pl.BlockSpec(
                    (8, 2, 1, 8, 128),
                    index_map=lambda i: (
                        (step0 + i) // _BT,
                        0,
                        (step0 + i) % _BT,
                        0,
                        0,
                    ),
                )
            ],
            core_axis_name="subcore",
            dimension_semantics=(pltpu.PARALLEL,),
        )(i_hbm, o_hbm)

    out_phys = gather_kernel(table, idx_phys)
    return out_phys.transpose(2, 4, 0, 1, 3).reshape(_BATCH, _HIST, _EMBED)


# speedup vs baseline: 4.2815x; 1.0132x over previous
import jax
import jax.numpy as jnp
from jax import lax
from jax.experimental import pallas as pl
from jax.experimental.pallas import tpu as pltpu
from jax.experimental.pallas import tpu_sc as plsc

_BATCH = 16384
_HIST = 200
_EMBED = 16
_N = _BATCH * _HIST

_HT = _HIST // 8
_BT = _BATCH // 128
_TILES = _HT * _BT
_TPS = 2


def kernel(pitch_type, table):
    idx_phys = (
        pitch_type.T.reshape(_HT, 8, _BT, 128)
        .transpose(0, 2, 1, 3)
        .reshape(_TILES, 1024)
        .astype(jnp.int32)
    )

    vector_mesh = plsc.VectorSubcoreMesh(
        core_axis_name="core", subcore_axis_name="subcore"
    )

    @pl.kernel(
        out_type=jax.ShapeDtypeStruct((_HIST, 2, _BT, 8, 128), jnp.float32),
        mesh=vector_mesh,
        scratch_types=[pltpu.VMEM((_TPS * 1024, _EMBED), jnp.float32)],
        compiler_params=pltpu.CompilerParams(
            use_tc_tiling_on_sc=False, needs_layout_passes=False
        ),
    )
    def gather_kernel(x_hbm, i_hbm, o_hbm, rows):
        core_id = lax.axis_index("core")
        steps_per_core = _TILES // _TPS // 2
        step0 = core_id * steps_per_core
        iota16 = lax.iota(jnp.int32, 16)
        cols = [jnp.full((16,), c, jnp.int32) for c in range(_EMBED)]

        def body(i_vmem, o_vmem):
            for k in range(_TPS):
                pltpu.sync_copy(
                    x_hbm.at[i_vmem.at[k]],
                    rows.at[pl.ds(k * 1024, 1024)],
                )

            def g_body(gg, carry):
                for u in range(4):
                    g = gg * 4 + u
                    k = g // 64
                    h = (g % 64) // 8
                    b0 = (g % 8) * 16
                    row_ids = g * 16 + iota16
                    for c in range(_EMBED):
                        vec = plsc.load_gather(rows, [row_ids, cols[c]])
                        o_vmem[h, c // 8, k, c % 8, pl.ds(b0, 16)] = vec
                return carry

            lax.fori_loop(0, _TPS * 64 // 4, g_body, 0)

        pltpu.emit_pipeline(
            body,
            grid=(steps_per_core,),
            in_specs=[
                pl.BlockSpec(
                    (_TPS, 1024), index_map=lambda i: (step0 + i, 0)
                )
            ],
            out_specs=[
                pl.BlockSpec(
                    (8, 2, _TPS, 8, 128),
                    index_map=lambda i: (
                        (step0 + i) // (_BT // _TPS),
                        0,
                        (step0 + i) % (_BT // _TPS),
                        0,
                        0,
                    ),
                )
            ],
            core_axis_name="subcore",
            dimension_semantics=(pltpu.PARALLEL,),
        )(i_hbm, o_hbm)

    out_phys = gather_kernel(table, idx_phys)
    return out_phys.transpose(2, 4, 0, 1, 3).reshape(_BATCH, _HIST, _EMBED)
